# baseline (speedup 1.0000x reference)
import functools

import jax
import jax.numpy as jnp
from jax import lax
from jax.experimental import pallas as pl
from jax.experimental.pallas import tpu as pltpu
from jax.experimental.pallas import tpu_sc as plsc


def _body(H, W, D, x_hbm, h_hbm, w_hbm, o_hbm, htab, wtab, buf):
    c = lax.axis_index("c")
    s = lax.axis_index("s")
    wid = s * 2 + c
    pltpu.sync_copy(h_hbm.at[pl.ds(0, H)], htab)
    pltpu.sync_copy(w_hbm.at[pl.ds(0, W)], wtab)
    base = wid * (H * W)
    nchunks = D // 16

    def body_h(h, carry):
        pltpu.sync_copy(x_hbm.at[pl.ds(base + h * W, W)], buf)

        def body_w(w, carry2):
            for cc in range(nchunks):
                off = cc * 16
                buf[w, pl.ds(off, 16)] = (
                    buf[w, pl.ds(off, 16)]
                    + htab[h, pl.ds(off, 16)]
                    + wtab[w, pl.ds(off, 16)])
            return carry2

        lax.fori_loop(0, W, body_w, 0)
        pltpu.sync_copy(buf, o_hbm.at[pl.ds(base + h * W, W)])
        return carry

    lax.fori_loop(0, H, body_h, 0)


def kernel(x, h_table, w_table):
    B, H, W, D = x.shape
    mesh = plsc.VectorSubcoreMesh(core_axis_name="c", subcore_axis_name="s")
    kern = pl.kernel(
        functools.partial(_body, H, W, D),
        out_type=jax.ShapeDtypeStruct((B * H * W, D), jnp.float32),
        mesh=mesh,
        scratch_types=[
            pltpu.VMEM((H, D), jnp.float32),
            pltpu.VMEM((W, D), jnp.float32),
            pltpu.VMEM((W, D), jnp.float32),
        ],
    )
    out = kern(x.reshape(B * H * W, D), h_table, w_table)
    return out.reshape(B, H, W, D)


# speedup vs baseline: 6.3141x; 6.3141x over previous
import jax
import jax.numpy as jnp
from jax.experimental import pallas as pl
from jax.experimental.pallas import tpu as pltpu

_NBUF = 4
_CHUNK = 2


def _body(x_hbm, h_ref, w_ref, o_hbm, s_ref, xbuf, obuf, insems, outsems):
    B = x_hbm.shape[0]
    n = B // _CHUNK
    s_ref[...] = h_ref[...][0][:, None, :] + w_ref[...][0][None, :, :]

    def in_copy(i):
        return pltpu.make_async_copy(
            x_hbm.at[pl.ds(i * _CHUNK, _CHUNK)], xbuf.at[i % _NBUF],
            insems.at[i % _NBUF])

    def out_copy(i):
        return pltpu.make_async_copy(
            obuf.at[i % _NBUF], o_hbm.at[pl.ds(i * _CHUNK, _CHUNK)],
            outsems.at[i % _NBUF])

    for i in range(min(_NBUF, n)):
        in_copy(i).start()
    for i in range(n):
        slot = i % _NBUF
        in_copy(i).wait()
        if i >= _NBUF:
            out_copy(i - _NBUF).wait()
        obuf[slot] = xbuf[slot] + s_ref[...][None]
        out_copy(i).start()
        if i + _NBUF < n:
            in_copy(i + _NBUF).start()
    for i in range(max(n - _NBUF, 0), n):
        out_copy(i).wait()


def kernel(x, h_table, w_table):
    B, H, W, D = x.shape
    return pl.pallas_call(
        _body,
        grid=(1,),
        in_specs=[
            pl.BlockSpec(memory_space=pl.ANY),
            pl.BlockSpec((1, H, D), lambda i: (0, 0, 0)),
            pl.BlockSpec((1, W, D), lambda i: (0, 0, 0)),
        ],
        out_specs=pl.BlockSpec(memory_space=pl.ANY),
        out_shape=jax.ShapeDtypeStruct((B, H, W, D), x.dtype),
        scratch_shapes=[
            pltpu.VMEM((H, W, D), x.dtype),
            pltpu.VMEM((_NBUF, _CHUNK, H, W, D), x.dtype),
            pltpu.VMEM((_NBUF, _CHUNK, H, W, D), x.dtype),
            pltpu.SemaphoreType.DMA((_NBUF,)),
            pltpu.SemaphoreType.DMA((_NBUF,)),
        ],
    )(x, h_table[None], w_table[None])


# speedup vs baseline: 6.3421x; 1.0044x over previous
import jax
import jax.numpy as jnp
from jax.experimental import pallas as pl
from jax.experimental.pallas import tpu as pltpu

_NBUF = 2
_CHUNK = 4


def _body(x_hbm, h_ref, w_ref, o_hbm, s_ref, xbuf, obuf, insems, outsems):
    B = x_hbm.shape[0]
    n = B // _CHUNK
    s_ref[...] = h_ref[...][0][:, None, :] + w_ref[...][0][None, :, :]

    def in_copy(i):
        return pltpu.make_async_copy(
            x_hbm.at[pl.ds(i * _CHUNK, _CHUNK)], xbuf.at[i % _NBUF],
            insems.at[i % _NBUF])

    def out_copy(i):
        return pltpu.make_async_copy(
            obuf.at[i % _NBUF], o_hbm.at[pl.ds(i * _CHUNK, _CHUNK)],
            outsems.at[i % _NBUF])

    for i in range(min(_NBUF, n)):
        in_copy(i).start()
    for i in range(n):
        slot = i % _NBUF
        in_copy(i).wait()
        if i >= _NBUF:
            out_copy(i - _NBUF).wait()
        obuf[slot] = xbuf[slot] + s_ref[...][None]
        out_copy(i).start()
        if i + _NBUF < n:
            in_copy(i + _NBUF).start()
    for i in range(max(n - _NBUF, 0), n):
        out_copy(i).wait()


def kernel(x, h_table, w_table):
    B, H, W, D = x.shape
    return pl.pallas_call(
        _body,
        grid=(1,),
        in_specs=[
            pl.BlockSpec(memory_space=pl.ANY),
            pl.BlockSpec((1, H, D), lambda i: (0, 0, 0)),
            pl.BlockSpec((1, W, D), lambda i: (0, 0, 0)),
        ],
        out_specs=pl.BlockSpec(memory_space=pl.ANY),
        out_shape=jax.ShapeDtypeStruct((B, H, W, D), x.dtype),
        scratch_shapes=[
            pltpu.VMEM((H, W, D), x.dtype),
            pltpu.VMEM((_NBUF, _CHUNK, H, W, D), x.dtype),
            pltpu.VMEM((_NBUF, _CHUNK, H, W, D), x.dtype),
            pltpu.SemaphoreType.DMA((_NBUF,)),
            pltpu.SemaphoreType.DMA((_NBUF,)),
        ],
    )(x, h_table[None], w_table[None])


# speedup vs baseline: 6.4881x; 1.0230x over previous
import jax
import jax.numpy as jnp
from jax.experimental import pallas as pl
from jax.experimental.pallas import tpu as pltpu

_NBUF = 4
_CHUNK = 4


def _body(x_hbm, h_ref, w_ref, o_hbm, s_ref, xbuf, insems, outsems):
    B = x_hbm.shape[0]
    n = B // _CHUNK
    s_ref[...] = h_ref[...][0][:, None, :] + w_ref[...][0][None, :, :]

    def in_copy(i):
        return pltpu.make_async_copy(
            x_hbm.at[pl.ds(i * _CHUNK, _CHUNK)], xbuf.at[i % _NBUF],
            insems.at[i % _NBUF])

    def out_copy(i):
        return pltpu.make_async_copy(
            xbuf.at[i % _NBUF], o_hbm.at[pl.ds(i * _CHUNK, _CHUNK)],
            outsems.at[i % _NBUF])

    for i in range(min(_NBUF, n)):
        in_copy(i).start()
    for i in range(n):
        slot = i % _NBUF
        in_copy(i).wait()
        xbuf[slot] = xbuf[slot] + s_ref[...][None]
        out_copy(i).start()
        nxt = i + _NBUF
        if nxt < n:
            out_copy(i).wait()
            in_copy(nxt).start()
    for i in range(max(n - _NBUF, 0), n):
        out_copy(i).wait()


def kernel(x, h_table, w_table):
    B, H, W, D = x.shape
    return pl.pallas_call(
        _body,
        grid=(1,),
        in_specs=[
            pl.BlockSpec(memory_space=pl.ANY),
            pl.BlockSpec((1, H, D), lambda i: (0, 0, 0)),
            pl.BlockSpec((1, W, D), lambda i: (0, 0, 0)),
        ],
        out_specs=pl.BlockSpec(memory_space=pl.ANY),
        out_shape=jax.ShapeDtypeStruct((B, H, W, D), x.dtype),
        scratch_shapes=[
            pltpu.VMEM((H, W, D), x.dtype),
            pltpu.VMEM((_NBUF, _CHUNK, H, W, D), x.dtype),
            pltpu.SemaphoreType.DMA((_NBUF,)),
            pltpu.SemaphoreType.DMA((_NBUF,)),
        ],
    )(x, h_table[None], w_table[None])
